# baseline (speedup 1.0000x reference)
import functools

import jax
import jax.numpy as jnp
import numpy as np
from jax import lax
from jax.experimental import pallas as pl
from jax.experimental.pallas import tpu as pltpu
from jax.experimental.pallas import tpu_sc as plsc

_B = 16384
_D = 128
_TD = 64
_NTIME = 365
_NC = 2
_NS = 16
_NW = _NC * _NS
_BPW = _B // _NW
_CHUNK = 64
_NCHUNK = _BPW // _CHUNK

_PERM = np.zeros(_D, dtype=np.int32)
for _k in range(_D // 32):
    for _i in range(16):
        _PERM[32 * _k + 2 * _i] = 32 * _k + _i
        _PERM[32 * _k + 2 * _i + 1] = 32 * _k + 16 + _i

_mesh = plsc.VectorSubcoreMesh(core_axis_name="c", subcore_axis_name="s")

_buf_set = [
    pltpu.VMEM((_CHUNK, _D), jnp.float32),
    pltpu.VMEM((_CHUNK, _D), jnp.float32),
    pltpu.VMEM((_CHUNK, _D), jnp.float32),
    pltpu.VMEM((_CHUNK, _D), jnp.float32),
    pltpu.VMEM((_CHUNK, _D), jnp.int32),
]


@functools.partial(
    pl.kernel,
    mesh=_mesh,
    compiler_params=pltpu.CompilerParams(needs_layout_passes=False),
    out_type=jax.ShapeDtypeStruct((_B,), jnp.float32),
    scratch_types=[
        pltpu.VMEM((_BPW,), jnp.int32),
        pltpu.VMEM((_BPW,), jnp.int32),
        pltpu.VMEM((_BPW,), jnp.int32),
        pltpu.VMEM((_BPW,), jnp.int32),
        _buf_set,
        _buf_set,
        pltpu.VMEM((_NTIME * _TD,), jnp.float32),
        pltpu.VMEM((_BPW,), jnp.float32),
        pltpu.SemaphoreType.DMA,
        pltpu.SemaphoreType.DMA,
    ],
)
def _tsimple_sc(heads_hbm, rels_hbm, tails_hbm, dates_hbm,
                eh_hbm, et_hbm, rc_hbm, tf_hbm,
                out_hbm,
                idx_h, idx_r, idx_t, idx_d,
                bufs_a, bufs_b, timv, outb, sem_a, sem_b):
    wid = lax.axis_index("s") * _NC + lax.axis_index("c")
    base = wid * _BPW

    _prolog_scope = jax.named_scope("prolog"); _prolog_scope.__enter__()
    pltpu.sync_copy(heads_hbm.at[pl.ds(base, _BPW)], idx_h)
    pltpu.sync_copy(rels_hbm.at[pl.ds(base, _BPW)], idx_r)
    pltpu.sync_copy(tails_hbm.at[pl.ds(base, _BPW)], idx_t)
    pltpu.sync_copy(dates_hbm.at[pl.ds(base, _BPW)], idx_d)
    pltpu.sync_copy(tf_hbm, timv)

    def start_set(ci, bufs, sem):
        sl_c = pl.ds(ci * _CHUNK, _CHUNK)
        h1, t1, h2, t2, rc = bufs
        pltpu.async_copy(eh_hbm.at[idx_h.at[sl_c]], h1, sem)
        pltpu.async_copy(et_hbm.at[idx_t.at[sl_c]], t1, sem)
        pltpu.async_copy(eh_hbm.at[idx_t.at[sl_c]], h2, sem)
        pltpu.async_copy(et_hbm.at[idx_h.at[sl_c]], t2, sem)
        pltpu.async_copy(rc_hbm.at[idx_r.at[sl_c]], rc, sem)

    def wait_set(bufs, sem):
        h1, t1, h2, t2, rc = bufs
        for b in (h1, t1, h2, t2):
            pltpu.make_async_copy(eh_hbm.at[pl.ds(0, _CHUNK)], b, sem).wait()
        pltpu.make_async_copy(rc_hbm.at[pl.ds(0, _CHUNK)], rc, sem).wait()

    lane_iota = lax.iota(jnp.int32, 16)
    idx15 = jnp.full((16, 1), 15, jnp.int32)
    gdn = lax.GatherDimensionNumbers(
        offset_dims=(), collapsed_slice_dims=(0,), start_index_map=(0,))

    def _unpack16(words):
        return plsc.unpack(plsc.bitcast(words, jnp.bfloat16),
                           format=plsc.PackFormat.INTERLEAVED,
                           preferred_element_type=jnp.float32)

    def compute_set(ci, bufs):
        h1, t1, h2, t2, rc = bufs
        for g in range(_CHUNK // 16):
            dvec = idx_d[pl.ds(ci * _CHUNK + g * 16, 16)]

            def e_body(l, res):
                e = g * 16 + l
                ddv = lax.gather(
                    dvec, jnp.full((16, 1), l, jnp.int32),
                    dimension_numbers=gdn, slice_sizes=(1,),
                    mode=lax.GatherScatterMode.PROMISE_IN_BOUNDS)
                tbase = ddv * _TD + lane_iota
                acc = jnp.zeros((16,), jnp.float32)
                for k2 in range(_D // 32):
                    r1p = _unpack16(rc[e, pl.ds(16 * k2, 16)])
                    r2p = _unpack16(rc[e, pl.ds(_TD + 16 * k2, 16)])
                    if k2 < _TD // 32:
                        tvp = (plsc.load_gather(timv, [tbase + 32 * k2]),
                               plsc.load_gather(timv, [tbase + 32 * k2 + 16]))
                    for h in range(2):
                        sl = pl.ds(k2 * 32 + h * 16, 16)
                        term = h1[e, sl] * r1p[h] * t1[e, sl] \
                            + h2[e, sl] * r2p[h] * t2[e, sl]
                        if k2 < _TD // 32:
                            term = term * tvp[h]
                        acc = acc + term
                csum = jnp.cumsum(acc)
                total = lax.gather(
                    csum, idx15, dimension_numbers=gdn, slice_sizes=(1,),
                    mode=lax.GatherScatterMode.PROMISE_IN_BOUNDS)
                return jnp.where(lane_iota == l, total, res)

            res = lax.fori_loop(0, 16, e_body, jnp.zeros((16,), jnp.float32))
            outb[pl.ds(ci * _CHUNK + g * 16, 16)] = res * 0.5

    _prolog_scope.__exit__(None, None, None)
    n2 = _NCHUNK // 2
    with jax.named_scope("start0"):
        start_set(0, bufs_a, sem_a)

    def pair_body(cj, carry):
        ci0 = 2 * cj
        with jax.named_scope("startB"):
            start_set(ci0 + 1, bufs_b, sem_b)
        with jax.named_scope("waitA"):
            wait_set(bufs_a, sem_a)
        with jax.named_scope("computeA"):
            compute_set(ci0, bufs_a)

        @pl.when(cj < n2 - 1)
        def _():
            with jax.named_scope("startA"):
                start_set(ci0 + 2, bufs_a, sem_a)

        with jax.named_scope("waitB"):
            wait_set(bufs_b, sem_b)
        with jax.named_scope("computeB"):
            compute_set(ci0 + 1, bufs_b)
        return carry

    lax.fori_loop(0, n2, pair_body, 0)
    pltpu.sync_copy(outb, out_hbm.at[pl.ds(base, _BPW)])


def _pack_words(tbl):
    n, w = tbl.shape
    shuffled = tbl[:, jnp.asarray(_PERM[:w])].astype(jnp.bfloat16)
    return jax.lax.bitcast_convert_type(
        shuffled.reshape(n, w // 2, 2), jnp.int32)


def kernel(heads, rels, tails, dates, ent_embs_h, ent_embs_t,
           rel_embs_f, rel_embs_i, tim_embs_f):
    rc = jnp.concatenate([_pack_words(rel_embs_f), _pack_words(rel_embs_i)],
                         axis=1)
    return _tsimple_sc(heads, rels, tails, dates, ent_embs_h, ent_embs_t,
                       rc, tim_embs_f.reshape(-1))


# speedup vs baseline: 1.0157x; 1.0157x over previous
import functools

import jax
import jax.numpy as jnp
import numpy as np
from jax import lax
from jax.experimental import pallas as pl
from jax.experimental.pallas import tpu as pltpu
from jax.experimental.pallas import tpu_sc as plsc

_B = 16384
_D = 128
_TD = 64
_NTIME = 365
_NC = 2
_NS = 16
_NW = _NC * _NS
_BPW = _B // _NW
_CHUNK = 64
_NCHUNK = _BPW // _CHUNK

_PERM = np.zeros(_D, dtype=np.int32)
for _k in range(_D // 32):
    for _i in range(16):
        _PERM[32 * _k + 2 * _i] = 32 * _k + _i
        _PERM[32 * _k + 2 * _i + 1] = 32 * _k + 16 + _i

_mesh = plsc.VectorSubcoreMesh(core_axis_name="c", subcore_axis_name="s")

_buf_set = [
    pltpu.VMEM((_CHUNK, _D), jnp.float32),
    pltpu.VMEM((_CHUNK, _D), jnp.float32),
    pltpu.VMEM((_CHUNK, _D), jnp.float32),
    pltpu.VMEM((_CHUNK, _D), jnp.float32),
    pltpu.VMEM((_CHUNK, _D), jnp.int32),
]


@functools.partial(
    pl.kernel,
    mesh=_mesh,
    compiler_params=pltpu.CompilerParams(needs_layout_passes=False),
    out_type=jax.ShapeDtypeStruct((_B,), jnp.float32),
    scratch_types=[
        pltpu.VMEM((4, _BPW), jnp.int32),
        _buf_set,
        _buf_set,
        pltpu.VMEM((_NTIME * _TD,), jnp.float32),
        pltpu.VMEM((_BPW,), jnp.float32),
        pltpu.SemaphoreType.DMA,
        pltpu.SemaphoreType.DMA,
        pltpu.SemaphoreType.DMA,
    ],
)
def _tsimple_sc(idx_hbm, eh_hbm, et_hbm, rc_hbm, tf_hbm,
                out_hbm,
                idxv, bufs_a, bufs_b, timv, outb, sem_a, sem_b, sem_t):
    wid = lax.axis_index("s") * _NC + lax.axis_index("c")

    tim_cp = pltpu.async_copy(tf_hbm, timv, sem_t)
    pltpu.sync_copy(idx_hbm.at[wid], idxv)

    def start_set(ci, bufs, sem):
        sl_c = pl.ds(ci * _CHUNK, _CHUNK)
        h1, t1, h2, t2, rc = bufs
        pltpu.async_copy(eh_hbm.at[idxv.at[0, sl_c]], h1, sem)
        pltpu.async_copy(et_hbm.at[idxv.at[2, sl_c]], t1, sem)
        pltpu.async_copy(eh_hbm.at[idxv.at[2, sl_c]], h2, sem)
        pltpu.async_copy(et_hbm.at[idxv.at[0, sl_c]], t2, sem)
        pltpu.async_copy(rc_hbm.at[idxv.at[1, sl_c]], rc, sem)

    def wait_set(bufs, sem):
        h1, t1, h2, t2, rc = bufs
        for b in (h1, t1, h2, t2):
            pltpu.make_async_copy(eh_hbm.at[pl.ds(0, _CHUNK)], b, sem).wait()
        pltpu.make_async_copy(rc_hbm.at[pl.ds(0, _CHUNK)], rc, sem).wait()

    lane_iota = lax.iota(jnp.int32, 16)
    idx15 = jnp.full((16, 1), 15, jnp.int32)
    gdn = lax.GatherDimensionNumbers(
        offset_dims=(), collapsed_slice_dims=(0,), start_index_map=(0,))

    def _unpack16(words):
        return plsc.unpack(plsc.bitcast(words, jnp.bfloat16),
                           format=plsc.PackFormat.INTERLEAVED,
                           preferred_element_type=jnp.float32)

    def compute_set(ci, bufs):
        h1, t1, h2, t2, rc = bufs
        for g in range(_CHUNK // 16):
            dvec = idxv[3, pl.ds(ci * _CHUNK + g * 16, 16)]

            def e_body(l, res):
                e = g * 16 + l
                ddv = lax.gather(
                    dvec, jnp.full((16, 1), l, jnp.int32),
                    dimension_numbers=gdn, slice_sizes=(1,),
                    mode=lax.GatherScatterMode.PROMISE_IN_BOUNDS)
                tbase = ddv * _TD + lane_iota
                acc = jnp.zeros((16,), jnp.float32)
                for k2 in range(_D // 32):
                    r1p = _unpack16(rc[e, pl.ds(16 * k2, 16)])
                    r2p = _unpack16(rc[e, pl.ds(_TD + 16 * k2, 16)])
                    if k2 < _TD // 32:
                        tvp = (plsc.load_gather(timv, [tbase + 32 * k2]),
                               plsc.load_gather(timv, [tbase + 32 * k2 + 16]))
                    for h in range(2):
                        sl = pl.ds(k2 * 32 + h * 16, 16)
                        term = h1[e, sl] * r1p[h] * t1[e, sl] \
                            + h2[e, sl] * r2p[h] * t2[e, sl]
                        if k2 < _TD // 32:
                            term = term * tvp[h]
                        acc = acc + term
                csum = jnp.cumsum(acc)
                total = lax.gather(
                    csum, idx15, dimension_numbers=gdn, slice_sizes=(1,),
                    mode=lax.GatherScatterMode.PROMISE_IN_BOUNDS)
                return jnp.where(lane_iota == l, total, res)

            res = lax.fori_loop(0, 16, e_body, jnp.zeros((16,), jnp.float32))
            outb[pl.ds(ci * _CHUNK + g * 16, 16)] = res * 0.5

    n2 = _NCHUNK // 2
    start_set(0, bufs_a, sem_a)
    start_set(1, bufs_b, sem_b)
    tim_cp.wait()

    def pair_body(cj, carry):
        ci0 = 2 * cj
        wait_set(bufs_a, sem_a)
        compute_set(ci0, bufs_a)

        @pl.when(cj < n2 - 1)
        def _():
            start_set(ci0 + 2, bufs_a, sem_a)

        wait_set(bufs_b, sem_b)
        compute_set(ci0 + 1, bufs_b)

        @pl.when(cj < n2 - 1)
        def _():
            start_set(ci0 + 3, bufs_b, sem_b)

        return carry

    lax.fori_loop(0, n2, pair_body, 0)
    pltpu.sync_copy(outb, out_hbm.at[pl.ds(wid * _BPW, _BPW)])


def _pack_words(tbl):
    n, w = tbl.shape
    shuffled = tbl[:, jnp.asarray(_PERM[:w])].astype(jnp.bfloat16)
    return jax.lax.bitcast_convert_type(
        shuffled.reshape(n, w // 2, 2), jnp.int32)


def kernel(heads, rels, tails, dates, ent_embs_h, ent_embs_t,
           rel_embs_f, rel_embs_i, tim_embs_f):
    rc = jnp.concatenate([_pack_words(rel_embs_f), _pack_words(rel_embs_i)],
                         axis=1)
    idx = jnp.stack([heads, rels, tails, dates])
    idx = idx.reshape(4, _NW, _BPW).transpose(1, 0, 2)
    return _tsimple_sc(idx, ent_embs_h, ent_embs_t,
                       rc, tim_embs_f.reshape(-1))


# speedup vs baseline: 1.0917x; 1.0749x over previous
import functools

import jax
import jax.numpy as jnp
from jax import lax
from jax.experimental import pallas as pl
from jax.experimental.pallas import tpu as pltpu
from jax.experimental.pallas import tpu_sc as plsc

_B = 16384
_D = 128
_TD = 64
_NTIME = 365
_NC = 2
_NS = 16
_NW = _NC * _NS
_BPW = _B // _NW
_CHUNK = 64
_NCHUNK = _BPW // _CHUNK

_mesh = plsc.VectorSubcoreMesh(core_axis_name="c", subcore_axis_name="s")

_buf_set = [
    pltpu.VMEM((_CHUNK, _D), jnp.float32),
    pltpu.VMEM((_CHUNK, _D), jnp.float32),
    pltpu.VMEM((_CHUNK, _D), jnp.float32),
    pltpu.VMEM((_CHUNK, _D), jnp.float32),
    pltpu.VMEM((_CHUNK, _D), jnp.float32),
    pltpu.VMEM((_CHUNK, _D), jnp.float32),
]


@functools.partial(
    pl.kernel,
    mesh=_mesh,
    compiler_params=pltpu.CompilerParams(needs_layout_passes=False),
    out_type=jax.ShapeDtypeStruct((_B,), jnp.float32),
    scratch_types=[
        pltpu.VMEM((_BPW,), jnp.int32),
        pltpu.VMEM((_BPW,), jnp.int32),
        pltpu.VMEM((_BPW,), jnp.int32),
        pltpu.VMEM((_BPW,), jnp.int32),
        _buf_set,
        _buf_set,
        pltpu.VMEM((_NTIME * _TD,), jnp.float32),
        pltpu.VMEM((_BPW,), jnp.float32),
        pltpu.SemaphoreType.DMA,
        pltpu.SemaphoreType.DMA,
        pltpu.SemaphoreType.DMA,
    ],
)
def _tsimple_sc(heads_hbm, rels_hbm, tails_hbm, dates_hbm,
                eh_hbm, et_hbm, rf_hbm, ri_hbm, tf_hbm,
                out_hbm,
                idx_h, idx_r, idx_t, idx_d,
                bufs_a, bufs_b, timv, outb, sem_a, sem_b, sem_t):
    wid = lax.axis_index("s") * _NC + lax.axis_index("c")
    base = wid * _BPW

    tim_cp = pltpu.async_copy(tf_hbm, timv, sem_t)
    idx_cps = [
        pltpu.async_copy(heads_hbm.at[pl.ds(base, _BPW)], idx_h, sem_a),
        pltpu.async_copy(rels_hbm.at[pl.ds(base, _BPW)], idx_r, sem_a),
        pltpu.async_copy(tails_hbm.at[pl.ds(base, _BPW)], idx_t, sem_a),
        pltpu.async_copy(dates_hbm.at[pl.ds(base, _BPW)], idx_d, sem_a),
    ]
    for cp in idx_cps:
        cp.wait()

    def start_set(ci, bufs, sem):
        sl_c = pl.ds(ci * _CHUNK, _CHUNK)
        h1, r1, t1, h2, r2, t2 = bufs
        pltpu.async_copy(eh_hbm.at[idx_h.at[sl_c]], h1, sem)
        pltpu.async_copy(rf_hbm.at[idx_r.at[sl_c]], r1, sem)
        pltpu.async_copy(et_hbm.at[idx_t.at[sl_c]], t1, sem)
        pltpu.async_copy(eh_hbm.at[idx_t.at[sl_c]], h2, sem)
        pltpu.async_copy(ri_hbm.at[idx_r.at[sl_c]], r2, sem)
        pltpu.async_copy(et_hbm.at[idx_h.at[sl_c]], t2, sem)

    def wait_set(bufs, sem):
        for b in bufs:
            pltpu.make_async_copy(eh_hbm.at[pl.ds(0, _CHUNK)], b, sem).wait()

    lane_iota = lax.iota(jnp.int32, 16)
    idx15 = jnp.full((16, 1), 15, jnp.int32)
    gdn = lax.GatherDimensionNumbers(
        offset_dims=(), collapsed_slice_dims=(0,), start_index_map=(0,))

    def compute_set(ci, bufs):
        h1, r1, t1, h2, r2, t2 = bufs
        for g in range(_CHUNK // 16):
            dvec = idx_d[pl.ds(ci * _CHUNK + g * 16, 16)]

            def e_body(l, res):
                e = g * 16 + l
                ddv = lax.gather(
                    dvec, jnp.full((16, 1), l, jnp.int32),
                    dimension_numbers=gdn, slice_sizes=(1,),
                    mode=lax.GatherScatterMode.PROMISE_IN_BOUNDS)
                tbase = ddv * _TD + lane_iota
                acc = jnp.zeros((16,), jnp.float32)
                for k in range(_D // 16):
                    sl = pl.ds(k * 16, 16)
                    term = h1[e, sl] * r1[e, sl] * t1[e, sl] \
                        + h2[e, sl] * r2[e, sl] * t2[e, sl]
                    if k < _TD // 16:
                        term = term * plsc.load_gather(timv, [tbase + 16 * k])
                    acc = acc + term
                csum = jnp.cumsum(acc)
                total = lax.gather(
                    csum, idx15, dimension_numbers=gdn, slice_sizes=(1,),
                    mode=lax.GatherScatterMode.PROMISE_IN_BOUNDS)
                return jnp.where(lane_iota == l, total, res)

            res = lax.fori_loop(0, 16, e_body, jnp.zeros((16,), jnp.float32))
            outb[pl.ds(ci * _CHUNK + g * 16, 16)] = res * 0.5

    n2 = _NCHUNK // 2
    start_set(0, bufs_a, sem_a)
    start_set(1, bufs_b, sem_b)
    tim_cp.wait()

    def pair_body(cj, carry):
        ci0 = 2 * cj
        wait_set(bufs_a, sem_a)
        compute_set(ci0, bufs_a)

        @pl.when(cj < n2 - 1)
        def _():
            start_set(ci0 + 2, bufs_a, sem_a)

        wait_set(bufs_b, sem_b)
        compute_set(ci0 + 1, bufs_b)

        @pl.when(cj < n2 - 1)
        def _():
            start_set(ci0 + 3, bufs_b, sem_b)

        return carry

    lax.fori_loop(0, n2, pair_body, 0)
    pltpu.sync_copy(outb, out_hbm.at[pl.ds(base, _BPW)])


def kernel(heads, rels, tails, dates, ent_embs_h, ent_embs_t,
           rel_embs_f, rel_embs_i, tim_embs_f):
    return _tsimple_sc(heads, rels, tails, dates, ent_embs_h, ent_embs_t,
                       rel_embs_f, rel_embs_i, tim_embs_f.reshape(-1))
